# baseline (speedup 1.0000x reference)
import functools

import jax
import jax.numpy as jnp
from jax import lax
from jax.experimental import pallas as pl
from jax.experimental.pallas import tpu as pltpu
from jax.experimental.pallas import tpu_sc as plsc

N_NODES = 10000
N_EDGES = 320000
D_IN = 128
D_HID = 64
D_OUT = 128

NC = 2
NS = 16
NW = NC * NS
CHUNK = 128
EDGES_PER_TILE = 10240
CHUNKS = EDGES_PER_TILE // CHUNK
E_PAD = NW * EDGES_PER_TILE
N_PAD = 10240
ROWS_PER_TILE = N_PAD // NS
ZCHUNKS = ROWS_PER_TILE // CHUNK
ROW_BLOCK = 1280
N_ROW_BLOCKS = N_PAD // ROW_BLOCK


def _mesh():
    return plsc.VectorSubcoreMesh(core_axis_name="c", subcore_axis_name="s",
                                  num_cores=NC, num_subcores=NS)


@functools.partial(
    pl.kernel,
    out_type=jax.ShapeDtypeStruct((NC, N_PAD), jnp.float32),
    mesh=_mesh(),
    scratch_types=[
        pltpu.VMEM((CHUNKS, CHUNK), jnp.int32),
        pltpu.VMEM((CHUNK,), jnp.float32),
        pltpu.VMEM((ROWS_PER_TILE,), jnp.float32),
        pltpu.VMEM_SHARED((N_PAD,), jnp.float32),
    ],
    compiler_params=pltpu.CompilerParams(use_tc_tiling_on_sc=False),
)
def _count_kernel(dst_hbm, cnt_hbm, idx_v, ones_v, tmp_v, acc_s):
    c = lax.axis_index("c")
    s = lax.axis_index("s")
    wid = c * NS + s
    pltpu.sync_copy(dst_hbm.at[wid], idx_v)
    for i in range(CHUNK // 16):
        ones_v[pl.ds(i * 16, 16)] = jnp.ones((16,), jnp.float32)

    def zero_body(i, carry):
        tmp_v[pl.ds(i * 16, 16)] = jnp.zeros((16,), jnp.float32)
        return carry

    lax.fori_loop(0, ROWS_PER_TILE // 16, zero_body, 0)
    tile_rows = pl.ds(s * ROWS_PER_TILE, ROWS_PER_TILE)
    pltpu.sync_copy(tmp_v, acc_s.at[tile_rows])
    plsc.subcore_barrier()

    def chunk_body(j, carry):
        pltpu.sync_copy(ones_v, acc_s.at[idx_v.at[j]], add=True)
        return carry

    lax.fori_loop(0, CHUNKS, chunk_body, 0)
    plsc.subcore_barrier()
    pltpu.sync_copy(acc_s.at[tile_rows], tmp_v)
    pltpu.sync_copy(tmp_v, cnt_hbm.at[c, tile_rows])


def _make_scatter(D, chunk):
    chunks = EDGES_PER_TILE // chunk
    assert EDGES_PER_TILE % chunk == 0 and chunks % 2 == 0
    @functools.partial(
        pl.kernel,
        out_type=jax.ShapeDtypeStruct((NC, N_PAD, D), jnp.float32),
        mesh=_mesh(),
        scratch_types=[
            pltpu.VMEM((chunks, chunk), jnp.int32),
            pltpu.VMEM((chunks, chunk), jnp.int32),
            pltpu.VMEM((chunk, D), jnp.float32),
            pltpu.VMEM((chunk, D), jnp.float32),
            pltpu.VMEM_SHARED((N_PAD, D), jnp.float32),
            pltpu.SemaphoreType.DMA,
            pltpu.SemaphoreType.DMA,
        ],
        compiler_params=pltpu.CompilerParams(use_tc_tiling_on_sc=False),
    )
    def _scatter(src_hbm, dst_hbm, xh_hbm, out_hbm,
                 srcv, dstv, rows0, rows1, acc_s, sem0, sem1):
        c = lax.axis_index("c")
        s = lax.axis_index("s")
        wid = c * NS + s
        pltpu.sync_copy(src_hbm.at[wid], srcv)
        pltpu.sync_copy(dst_hbm.at[wid], dstv)

        def zero_body(r, carry):
            for i in range(D // 16):
                rows0[r, pl.ds(i * 16, 16)] = jnp.zeros((16,), jnp.float32)
            return carry

        lax.fori_loop(0, chunk, zero_body, 0)
        zchunks = ROWS_PER_TILE // chunk
        ztail = ROWS_PER_TILE % chunk

        def rz(z):
            return pl.ds(s * ROWS_PER_TILE + z * chunk, chunk)

        for z in range(zchunks):
            pltpu.async_copy(rows0, acc_s.at[rz(z)], sem0)
        if ztail:
            pltpu.async_copy(rows0.at[pl.ds(0, ztail)],
                             acc_s.at[pl.ds(s * ROWS_PER_TILE
                                            + zchunks * chunk, ztail)], sem0)
        for z in range(zchunks):
            pltpu.make_async_copy(rows0, acc_s.at[rz(z)], sem0).wait()
        if ztail:
            pltpu.make_async_copy(rows0.at[pl.ds(0, ztail)],
                                  acc_s.at[pl.ds(s * ROWS_PER_TILE
                                                 + zchunks * chunk, ztail)],
                                  sem0).wait()
        plsc.subcore_barrier()

        pltpu.async_copy(xh_hbm.at[srcv.at[0]], rows0, sem0)
        pltpu.async_copy(xh_hbm.at[srcv.at[1]], rows1, sem1)

        def pair_body(p, carry):
            j0 = p * 2
            pltpu.make_async_copy(xh_hbm.at[srcv.at[j0]], rows0, sem0).wait()
            pltpu.sync_copy(rows0, acc_s.at[dstv.at[j0]], add=True)

            @pl.when(j0 + 2 < chunks)
            def _():
                pltpu.async_copy(xh_hbm.at[srcv.at[j0 + 2]], rows0, sem0)

            pltpu.make_async_copy(
                xh_hbm.at[srcv.at[j0 + 1]], rows1, sem1).wait()
            pltpu.sync_copy(rows1, acc_s.at[dstv.at[j0 + 1]], add=True)

            @pl.when(j0 + 3 < chunks)
            def _():
                pltpu.async_copy(xh_hbm.at[srcv.at[j0 + 3]], rows1, sem1)

            return carry

        lax.fori_loop(0, chunks // 2, pair_body, 0)

        plsc.subcore_barrier()
        def rzo(z, size):
            return pl.ds(s * ROWS_PER_TILE + z * chunk, size)

        nz = zchunks + (1 if ztail else 0)
        for z in range(nz):
            size = chunk if z < zchunks else ztail
            buf = rows0 if z % 2 == 0 else rows1
            bufs = buf if size == chunk else buf.at[pl.ds(0, size)]
            pltpu.sync_copy(acc_s.at[rzo(z, size)], bufs)
            pltpu.sync_copy(bufs, out_hbm.at[c, rzo(z, size)])

    return _scatter


CHUNK_HID = 256
CHUNK_OUT = 64
_scatter_hid = _make_scatter(D_HID, CHUNK_HID)
_scatter_out = _make_scatter(D_OUT, CHUNK_OUT)


def _tc1_body(cnt_ref, x_ref, w_ref, xh_ref, dinv_ref):
    cnt = cnt_ref[0] + cnt_ref[1]
    dinv = lax.rsqrt(cnt + 1.0)
    h = jnp.dot(x_ref[...], w_ref[...], preferred_element_type=jnp.float32)
    xh_ref[...] = h * dinv
    dinv_ref[...] = dinv


def _tc1(cnt2, x_p, W1):
    return pl.pallas_call(
        _tc1_body,
        grid=(N_ROW_BLOCKS,),
        in_specs=[
            pl.BlockSpec((NC, ROW_BLOCK, 1), lambda i: (0, i, 0)),
            pl.BlockSpec((ROW_BLOCK, D_IN), lambda i: (i, 0)),
            pl.BlockSpec((D_IN, D_HID), lambda i: (0, 0)),
        ],
        out_specs=[
            pl.BlockSpec((ROW_BLOCK, D_HID), lambda i: (i, 0)),
            pl.BlockSpec((ROW_BLOCK, 1), lambda i: (i, 0)),
        ],
        out_shape=[
            jax.ShapeDtypeStruct((N_PAD, D_HID), jnp.float32),
            jax.ShapeDtypeStruct((N_PAD, 1), jnp.float32),
        ],
    )(cnt2, x_p, W1)


def _tc2_body(acc_ref, xh_ref, dinv_ref, b_ref, w_ref, out_ref):
    dinv = dinv_ref[...]
    z = (acc_ref[0] + acc_ref[1] + xh_ref[...]) * dinv + b_ref[...]
    a = jnp.maximum(z, 0.0)
    out_ref[...] = jnp.dot(
        a, w_ref[...], preferred_element_type=jnp.float32) * dinv


def _tc2(acc1, xh1, dinv, b1, W2):
    return pl.pallas_call(
        _tc2_body,
        grid=(N_ROW_BLOCKS,),
        in_specs=[
            pl.BlockSpec((NC, ROW_BLOCK, D_HID), lambda i: (0, i, 0)),
            pl.BlockSpec((ROW_BLOCK, D_HID), lambda i: (i, 0)),
            pl.BlockSpec((ROW_BLOCK, 1), lambda i: (i, 0)),
            pl.BlockSpec((1, D_HID), lambda i: (0, 0)),
            pl.BlockSpec((D_HID, D_OUT), lambda i: (0, 0)),
        ],
        out_specs=pl.BlockSpec((ROW_BLOCK, D_OUT), lambda i: (i, 0)),
        out_shape=jax.ShapeDtypeStruct((N_PAD, D_OUT), jnp.float32),
    )(acc1, xh1, dinv, b1, W2)


def _tc3_body(acc_ref, xh_ref, dinv_ref, b_ref, out_ref):
    out_ref[...] = (acc_ref[0] + acc_ref[1] + xh_ref[...]) * dinv_ref[...] \
        + b_ref[...]


def _tc3(acc2, xh2, dinv, b2):
    return pl.pallas_call(
        _tc3_body,
        grid=(N_ROW_BLOCKS,),
        in_specs=[
            pl.BlockSpec((NC, ROW_BLOCK, D_OUT), lambda i: (0, i, 0)),
            pl.BlockSpec((ROW_BLOCK, D_OUT), lambda i: (i, 0)),
            pl.BlockSpec((ROW_BLOCK, 1), lambda i: (i, 0)),
            pl.BlockSpec((1, D_OUT), lambda i: (0, 0)),
        ],
        out_specs=pl.BlockSpec((ROW_BLOCK, D_OUT), lambda i: (i, 0)),
        out_shape=jax.ShapeDtypeStruct((N_PAD, D_OUT), jnp.float32),
    )(acc2, xh2, dinv, b2)


def kernel(x, edge_index, W1, b1, W2, b2):
    src = edge_index[0].astype(jnp.int32)
    dst = edge_index[1].astype(jnp.int32)
    npad_e = E_PAD - N_EDGES
    pad_src = jnp.full((npad_e,), N_NODES, jnp.int32)
    pad_dst = N_NODES + (jnp.arange(npad_e, dtype=jnp.int32)
                         % (N_PAD - N_NODES))
    src_p = jnp.concatenate([src, pad_src])
    dst_p = jnp.concatenate([dst, pad_dst])
    def esh(chunk):
        return (NW, EDGES_PER_TILE // chunk, chunk)

    x_p = jnp.zeros((N_PAD, D_IN), jnp.float32).at[:N_NODES].set(x)

    cnt2 = _count_kernel(dst_p.reshape(esh(CHUNK)))
    xh1, dinv = _tc1(cnt2.reshape(NC, N_PAD, 1), x_p, W1)
    acc1 = _scatter_hid(src_p.reshape(esh(CHUNK_HID)),
                        dst_p.reshape(esh(CHUNK_HID)), xh1)
    xh2 = _tc2(acc1, xh1, dinv, b1.reshape(1, D_HID), W2)
    acc2 = _scatter_out(src_p.reshape(esh(CHUNK_OUT)),
                        dst_p.reshape(esh(CHUNK_OUT)), xh2)
    out = _tc3(acc2, xh2, dinv, b2.reshape(1, D_OUT))
    return out[:N_NODES]


# speedup vs baseline: 2.6802x; 2.6802x over previous
import functools

import jax
import jax.numpy as jnp
from jax import lax
from jax.experimental import pallas as pl
from jax.experimental.pallas import tpu as pltpu
from jax.experimental.pallas import tpu_sc as plsc

N_NODES = 10000
N_EDGES = 320000
D_IN = 128
D_HID = 64
D_OUT = 128

NC = 2
NS = 16
NW = NC * NS
CHUNK = 128
EDGES_PER_TILE = 10240
CHUNKS = EDGES_PER_TILE // CHUNK
E_PAD = NW * EDGES_PER_TILE
N_PAD = 10240
ROWS_PER_TILE = N_PAD // NS
ZCHUNKS = ROWS_PER_TILE // CHUNK
ROW_BLOCK = 1280
N_ROW_BLOCKS = N_PAD // ROW_BLOCK


def _mesh():
    return plsc.VectorSubcoreMesh(core_axis_name="c", subcore_axis_name="s",
                                  num_cores=NC, num_subcores=NS)


@functools.partial(
    pl.kernel,
    out_type=jax.ShapeDtypeStruct((NC, N_PAD), jnp.float32),
    mesh=_mesh(),
    scratch_types=[
        pltpu.VMEM((CHUNKS, CHUNK), jnp.int32),
        pltpu.VMEM((CHUNK,), jnp.float32),
        pltpu.VMEM((ROWS_PER_TILE,), jnp.float32),
        pltpu.VMEM_SHARED((N_PAD,), jnp.float32),
    ],
    compiler_params=pltpu.CompilerParams(use_tc_tiling_on_sc=False),
)
def _count_kernel(dst_hbm, cnt_hbm, idx_v, ones_v, tmp_v, acc_s):
    c = lax.axis_index("c")
    s = lax.axis_index("s")
    wid = c * NS + s
    pltpu.sync_copy(dst_hbm.at[wid], idx_v)
    for i in range(CHUNK // 16):
        ones_v[pl.ds(i * 16, 16)] = jnp.ones((16,), jnp.float32)

    def zero_body(i, carry):
        tmp_v[pl.ds(i * 16, 16)] = jnp.zeros((16,), jnp.float32)
        return carry

    lax.fori_loop(0, ROWS_PER_TILE // 16, zero_body, 0)
    tile_rows = pl.ds(s * ROWS_PER_TILE, ROWS_PER_TILE)
    pltpu.sync_copy(tmp_v, acc_s.at[tile_rows])
    plsc.subcore_barrier()

    def chunk_body(j, carry):
        pltpu.sync_copy(ones_v, acc_s.at[idx_v.at[j]], add=True)
        return carry

    lax.fori_loop(0, CHUNKS, chunk_body, 0)
    plsc.subcore_barrier()
    pltpu.sync_copy(acc_s.at[tile_rows], tmp_v)
    pltpu.sync_copy(tmp_v, cnt_hbm.at[c, tile_rows])


def _make_scatter(D, chunk):
    chunks = EDGES_PER_TILE // chunk
    assert EDGES_PER_TILE % chunk == 0 and chunks % 2 == 0
    @functools.partial(
        pl.kernel,
        out_type=jax.ShapeDtypeStruct((NC, N_PAD, D), jnp.float32),
        mesh=_mesh(),
        scratch_types=[
            pltpu.VMEM((chunks, chunk), jnp.int32),
            pltpu.VMEM((chunks, chunk), jnp.int32),
            pltpu.VMEM((chunk, D), jnp.float32),
            pltpu.VMEM((chunk, D), jnp.float32),
            pltpu.VMEM_SHARED((N_PAD, D), jnp.float32),
            pltpu.SemaphoreType.DMA,
            pltpu.SemaphoreType.DMA,
        ],
        compiler_params=pltpu.CompilerParams(use_tc_tiling_on_sc=False),
    )
    def _scatter(src_hbm, dst_hbm, xh_hbm, out_hbm,
                 srcv, dstv, rows0, rows1, acc_s, sem0, sem1):
        c = lax.axis_index("c")
        s = lax.axis_index("s")
        wid = c * NS + s
        pltpu.sync_copy(src_hbm.at[wid], srcv)
        pltpu.sync_copy(dst_hbm.at[wid], dstv)

        def zero_body(r, carry):
            for i in range(D // 16):
                rows0[r, pl.ds(i * 16, 16)] = jnp.zeros((16,), jnp.float32)
            return carry

        lax.fori_loop(0, chunk, zero_body, 0)
        zchunks = ROWS_PER_TILE // chunk
        ztail = ROWS_PER_TILE % chunk

        def rz(z):
            return pl.ds(s * ROWS_PER_TILE + z * chunk, chunk)

        for z in range(zchunks):
            pltpu.async_copy(rows0, acc_s.at[rz(z)], sem0)
        if ztail:
            pltpu.async_copy(rows0.at[pl.ds(0, ztail)],
                             acc_s.at[pl.ds(s * ROWS_PER_TILE
                                            + zchunks * chunk, ztail)], sem0)
        for z in range(zchunks):
            pltpu.make_async_copy(rows0, acc_s.at[rz(z)], sem0).wait()
        if ztail:
            pltpu.make_async_copy(rows0.at[pl.ds(0, ztail)],
                                  acc_s.at[pl.ds(s * ROWS_PER_TILE
                                                 + zchunks * chunk, ztail)],
                                  sem0).wait()
        plsc.subcore_barrier()

        pltpu.async_copy(xh_hbm.at[srcv.at[0]], rows0, sem0)
        pltpu.async_copy(xh_hbm.at[srcv.at[1]], rows1, sem1)

        def pair_body(p, carry):
            j0 = p * 2
            pltpu.make_async_copy(xh_hbm.at[srcv.at[j0]], rows0, sem0).wait()
            pltpu.sync_copy(rows0, acc_s.at[dstv.at[j0]], add=True)

            @pl.when(j0 + 2 < chunks)
            def _():
                pltpu.async_copy(xh_hbm.at[srcv.at[j0 + 2]], rows0, sem0)

            pltpu.make_async_copy(
                xh_hbm.at[srcv.at[j0 + 1]], rows1, sem1).wait()
            pltpu.sync_copy(rows1, acc_s.at[dstv.at[j0 + 1]], add=True)

            @pl.when(j0 + 3 < chunks)
            def _():
                pltpu.async_copy(xh_hbm.at[srcv.at[j0 + 3]], rows1, sem1)

            return carry

        lax.fori_loop(0, chunks // 2, pair_body, 0)

        plsc.subcore_barrier()
        def rzo(z, size):
            return pl.ds(s * ROWS_PER_TILE + z * chunk, size)

        nz = zchunks + (1 if ztail else 0)
        for z in range(nz):
            size = chunk if z < zchunks else ztail
            buf = rows0 if z % 2 == 0 else rows1
            bufs = buf if size == chunk else buf.at[pl.ds(0, size)]
            pltpu.sync_copy(acc_s.at[rzo(z, size)], bufs)
            pltpu.sync_copy(bufs, out_hbm.at[c, rzo(z, size)])

    return _scatter


CHUNK_HID = 256
CHUNK_OUT = 64
_scatter_hid = _make_scatter(D_HID, CHUNK_HID)
_scatter_out = _make_scatter(D_OUT, CHUNK_OUT)


def _tc1_body(cnt_ref, x_ref, w_ref, xh_ref, dinv_ref):
    cnt = cnt_ref[0] + cnt_ref[1]
    dinv = lax.rsqrt(cnt + 1.0)
    h = jnp.dot(x_ref[...], w_ref[...], preferred_element_type=jnp.float32)
    xh_ref[...] = h * dinv
    dinv_ref[...] = dinv


def _tc1(cnt2, x_p, W1):
    return pl.pallas_call(
        _tc1_body,
        grid=(N_ROW_BLOCKS,),
        in_specs=[
            pl.BlockSpec((NC, ROW_BLOCK, 1), lambda i: (0, i, 0)),
            pl.BlockSpec((ROW_BLOCK, D_IN), lambda i: (i, 0)),
            pl.BlockSpec((D_IN, D_HID), lambda i: (0, 0)),
        ],
        out_specs=[
            pl.BlockSpec((ROW_BLOCK, D_HID), lambda i: (i, 0)),
            pl.BlockSpec((ROW_BLOCK, 1), lambda i: (i, 0)),
        ],
        out_shape=[
            jax.ShapeDtypeStruct((N_PAD, D_HID), jnp.float32),
            jax.ShapeDtypeStruct((N_PAD, 1), jnp.float32),
        ],
    )(cnt2, x_p, W1)


def _tc2_body(acc_ref, xh_ref, dinv_ref, b_ref, w_ref, out_ref):
    dinv = dinv_ref[...]
    z = (acc_ref[0] + acc_ref[1] + xh_ref[...]) * dinv + b_ref[...]
    a = jnp.maximum(z, 0.0)
    out_ref[...] = jnp.dot(
        a, w_ref[...], preferred_element_type=jnp.float32) * dinv


def _tc2(acc1, xh1, dinv, b1, W2):
    return pl.pallas_call(
        _tc2_body,
        grid=(N_ROW_BLOCKS,),
        in_specs=[
            pl.BlockSpec((NC, ROW_BLOCK, D_HID), lambda i: (0, i, 0)),
            pl.BlockSpec((ROW_BLOCK, D_HID), lambda i: (i, 0)),
            pl.BlockSpec((ROW_BLOCK, 1), lambda i: (i, 0)),
            pl.BlockSpec((1, D_HID), lambda i: (0, 0)),
            pl.BlockSpec((D_HID, D_OUT), lambda i: (0, 0)),
        ],
        out_specs=pl.BlockSpec((ROW_BLOCK, D_OUT), lambda i: (i, 0)),
        out_shape=jax.ShapeDtypeStruct((N_PAD, D_OUT), jnp.float32),
    )(acc1, xh1, dinv, b1, W2)


def _tc3_body(acc_ref, xh_ref, dinv_ref, b_ref, out_ref):
    out_ref[...] = (acc_ref[0] + acc_ref[1] + xh_ref[...]) * dinv_ref[...] \
        + b_ref[...]


def _tc3(acc2, xh2, dinv, b2):
    return pl.pallas_call(
        _tc3_body,
        grid=(N_ROW_BLOCKS,),
        in_specs=[
            pl.BlockSpec((NC, ROW_BLOCK, D_OUT), lambda i: (0, i, 0)),
            pl.BlockSpec((ROW_BLOCK, D_OUT), lambda i: (i, 0)),
            pl.BlockSpec((ROW_BLOCK, 1), lambda i: (i, 0)),
            pl.BlockSpec((1, D_OUT), lambda i: (0, 0)),
        ],
        out_specs=pl.BlockSpec((ROW_BLOCK, D_OUT), lambda i: (i, 0)),
        out_shape=jax.ShapeDtypeStruct((N_PAD, D_OUT), jnp.float32),
    )(acc2, xh2, dinv, b2)


def kernel(x, edge_index, W1, b1, W2, b2):
    src = edge_index[0].astype(jnp.int32)
    dst = edge_index[1].astype(jnp.int32)
    ppt = EDGES_PER_TILE - N_EDGES // NW
    padrows = N_NODES + (jnp.arange(ppt, dtype=jnp.int32)
                         % (N_PAD - N_NODES))
    pad_blk = jnp.broadcast_to(padrows, (NW, ppt))
    src_p = jnp.concatenate([src.reshape(NW, N_EDGES // NW), pad_blk], axis=1)
    dst_p = jnp.concatenate([dst.reshape(NW, N_EDGES // NW), pad_blk], axis=1)
    def esh(chunk):
        return (NW, EDGES_PER_TILE // chunk, chunk)

    x_p = jnp.zeros((N_PAD, D_IN), jnp.float32).at[:N_NODES].set(x)

    cnt2 = _count_kernel(dst_p.reshape(esh(CHUNK)))
    xh1, dinv = _tc1(cnt2.reshape(NC, N_PAD, 1), x_p, W1)
    acc1 = _scatter_hid(src_p.reshape(esh(CHUNK_HID)),
                        dst_p.reshape(esh(CHUNK_HID)), xh1)
    xh2 = _tc2(acc1, xh1, dinv, b1.reshape(1, D_HID), W2)
    acc2 = _scatter_out(src_p.reshape(esh(CHUNK_OUT)),
                        dst_p.reshape(esh(CHUNK_OUT)), xh2)
    out = _tc3(acc2, xh2, dinv, b2.reshape(1, D_OUT))
    return out[:N_NODES]


# speedup vs baseline: 3.0106x; 1.1233x over previous
import functools

import jax
import jax.numpy as jnp
from jax import lax
from jax.experimental import pallas as pl
from jax.experimental.pallas import tpu as pltpu
from jax.experimental.pallas import tpu_sc as plsc

N_NODES = 10000
N_EDGES = 320000
D_IN = 128
D_HID = 64
D_OUT = 128

NC = 2
NS = 16
NW = NC * NS
EPT = N_EDGES // NW
N_PAD = 10240
ROWS_PER_TILE = N_PAD // NS
CHUNK_CNT = 400
CHUNK_HID = 400
CHUNK_OUT = 80
ROW_BLOCK = 1000
N_ROW_BLOCKS = N_NODES // ROW_BLOCK


def _mesh():
    return plsc.VectorSubcoreMesh(core_axis_name="c", subcore_axis_name="s",
                                  num_cores=NC, num_subcores=NS)


@functools.partial(
    pl.kernel,
    out_type=jax.ShapeDtypeStruct((NC * N_PAD,), jnp.float32),
    mesh=_mesh(),
    scratch_types=[
        pltpu.VMEM((EPT,), jnp.int32),
        pltpu.VMEM((CHUNK_CNT,), jnp.float32),
        pltpu.VMEM((ROWS_PER_TILE,), jnp.float32),
        pltpu.VMEM_SHARED((N_PAD,), jnp.float32),
    ],
    compiler_params=pltpu.CompilerParams(use_tc_tiling_on_sc=False),
)
def _count_kernel(dst_hbm, cnt_hbm, idx_v, ones_v, tmp_v, acc_s):
    c = lax.axis_index("c")
    s = lax.axis_index("s")
    wid = c * NS + s
    pltpu.sync_copy(dst_hbm.at[wid], idx_v)

    def ones_body(i, carry):
        ones_v[pl.ds(i * 16, 16)] = jnp.ones((16,), jnp.float32)
        return carry

    lax.fori_loop(0, CHUNK_CNT // 16, ones_body, 0)

    def zero_body(i, carry):
        tmp_v[pl.ds(i * 16, 16)] = jnp.zeros((16,), jnp.float32)
        return carry

    lax.fori_loop(0, ROWS_PER_TILE // 16, zero_body, 0)
    tile_rows = pl.ds(s * ROWS_PER_TILE, ROWS_PER_TILE)
    pltpu.sync_copy(tmp_v, acc_s.at[tile_rows])
    plsc.subcore_barrier()

    def chunk_body(j, carry):
        pltpu.sync_copy(ones_v,
                        acc_s.at[idx_v.at[pl.ds(j * CHUNK_CNT, CHUNK_CNT)]],
                        add=True)
        return carry

    lax.fori_loop(0, EPT // CHUNK_CNT, chunk_body, 0)
    plsc.subcore_barrier()
    pltpu.sync_copy(acc_s.at[tile_rows], tmp_v)
    pltpu.sync_copy(tmp_v,
                    cnt_hbm.at[pl.ds(c * N_PAD + s * ROWS_PER_TILE,
                                     ROWS_PER_TILE)])


def _make_scatter(D, chunk):
    chunks = EPT // chunk
    assert EPT % chunk == 0 and chunks >= 3 and chunk % 8 == 0

    @functools.partial(
        pl.kernel,
        out_type=jax.ShapeDtypeStruct((NC, N_PAD, D), jnp.float32),
        mesh=_mesh(),
        scratch_types=[
            pltpu.VMEM((EPT,), jnp.int32),
            pltpu.VMEM((EPT,), jnp.int32),
            pltpu.VMEM((chunk, D), jnp.float32),
            pltpu.VMEM((chunk, D), jnp.float32),
            pltpu.VMEM_SHARED((N_PAD, D), jnp.float32),
            pltpu.SemaphoreType.DMA,
            pltpu.SemaphoreType.DMA,
        ],
        compiler_params=pltpu.CompilerParams(use_tc_tiling_on_sc=False),
    )
    def _scatter(src_hbm, dst_hbm, xh_hbm, out_hbm,
                 srcv, dstv, rows0, rows1, acc_s, sem0, sem1):
        c = lax.axis_index("c")
        s = lax.axis_index("s")
        wid = c * NS + s
        pltpu.sync_copy(src_hbm.at[wid], srcv)
        pltpu.sync_copy(dst_hbm.at[wid], dstv)

        def sidx(ref, j):
            return ref.at[pl.ds(j * chunk, chunk)]

        def zero_body(r, carry):
            for i in range(D // 16):
                rows0[r, pl.ds(i * 16, 16)] = jnp.zeros((16,), jnp.float32)
            return carry

        lax.fori_loop(0, chunk, zero_body, 0)
        zchunks = ROWS_PER_TILE // chunk
        ztail = ROWS_PER_TILE % chunk

        def zslices():
            out = []
            for z in range(zchunks):
                out.append((pl.ds(s * ROWS_PER_TILE + z * chunk, chunk), None))
            if ztail:
                out.append((pl.ds(s * ROWS_PER_TILE + zchunks * chunk, ztail),
                            ztail))
            return out

        for sl, size in zslices():
            buf = rows0 if size is None else rows0.at[pl.ds(0, size)]
            pltpu.async_copy(buf, acc_s.at[sl], sem0)
        for sl, size in zslices():
            buf = rows0 if size is None else rows0.at[pl.ds(0, size)]
            pltpu.make_async_copy(buf, acc_s.at[sl], sem0).wait()
        plsc.subcore_barrier()

        pltpu.async_copy(xh_hbm.at[sidx(srcv, 0)], rows0, sem0)
        pltpu.async_copy(xh_hbm.at[sidx(srcv, 1)], rows1, sem1)

        def pair_body(p, carry):
            j0 = p * 2
            pltpu.make_async_copy(
                xh_hbm.at[sidx(srcv, j0)], rows0, sem0).wait()
            pltpu.sync_copy(rows0, acc_s.at[sidx(dstv, j0)], add=True)

            @pl.when(j0 + 2 < chunks)
            def _():
                pltpu.async_copy(xh_hbm.at[sidx(srcv, j0 + 2)], rows0, sem0)

            pltpu.make_async_copy(
                xh_hbm.at[sidx(srcv, j0 + 1)], rows1, sem1).wait()
            pltpu.sync_copy(rows1, acc_s.at[sidx(dstv, j0 + 1)], add=True)

            @pl.when(j0 + 3 < chunks)
            def _():
                pltpu.async_copy(xh_hbm.at[sidx(srcv, j0 + 3)], rows1, sem1)

            return carry

        lax.fori_loop(0, chunks // 2, pair_body, 0)
        if chunks % 2:
            jl = chunks - 1
            pltpu.make_async_copy(
                xh_hbm.at[sidx(srcv, jl)], rows0, sem0).wait()
            pltpu.sync_copy(rows0, acc_s.at[sidx(dstv, jl)], add=True)

        plsc.subcore_barrier()
        zs = zslices()
        for z, (sl, size) in enumerate(zs):
            buf = rows0 if z % 2 == 0 else rows1
            bufv = buf if size is None else buf.at[pl.ds(0, size)]
            pltpu.sync_copy(acc_s.at[sl], bufv)
            pltpu.sync_copy(bufv, out_hbm.at[c, sl])

    return _scatter


_scatter_hid = _make_scatter(D_HID, CHUNK_HID)
_scatter_out = _make_scatter(D_OUT, CHUNK_OUT)


def _col(v):
    return jnp.transpose(v.reshape(1, -1))


def _tc1_body(cnt0_ref, cnt1_ref, x_ref, w_ref, xh_ref, dinv_ref):
    cnt = cnt0_ref[...] + cnt1_ref[...]
    dinv = lax.rsqrt(cnt + 1.0)
    dinv_ref[...] = dinv
    h = jnp.dot(x_ref[...], w_ref[...], preferred_element_type=jnp.float32)
    xh_ref[...] = h * _col(dinv)


def _tc1(cnt0, cnt1, x, W1):
    return pl.pallas_call(
        _tc1_body,
        grid=(N_ROW_BLOCKS,),
        in_specs=[
            pl.BlockSpec((1, 1, ROW_BLOCK), lambda i: (i, 0, 0)),
            pl.BlockSpec((1, 1, ROW_BLOCK), lambda i: (i, 0, 0)),
            pl.BlockSpec((ROW_BLOCK, D_IN), lambda i: (i, 0)),
            pl.BlockSpec((D_IN, D_HID), lambda i: (0, 0)),
        ],
        out_specs=[
            pl.BlockSpec((ROW_BLOCK, D_HID), lambda i: (i, 0)),
            pl.BlockSpec((1, 1, ROW_BLOCK), lambda i: (i, 0, 0)),
        ],
        out_shape=[
            jax.ShapeDtypeStruct((N_NODES, D_HID), jnp.float32),
            jax.ShapeDtypeStruct((N_ROW_BLOCKS, 1, ROW_BLOCK), jnp.float32),
        ],
    )(cnt0, cnt1, x, W1)


def _tc2_body(acc_ref, xh_ref, dinv_ref, b_ref, w_ref, out_ref):
    dinv = _col(dinv_ref[...])
    z = (acc_ref[0] + acc_ref[1] + xh_ref[...]) * dinv + b_ref[...]
    a = jnp.maximum(z, 0.0)
    out_ref[...] = jnp.dot(
        a, w_ref[...], preferred_element_type=jnp.float32) * dinv


def _tc2(acc1, xh1, dinv, b1, W2):
    return pl.pallas_call(
        _tc2_body,
        grid=(N_ROW_BLOCKS,),
        in_specs=[
            pl.BlockSpec((NC, ROW_BLOCK, D_HID), lambda i: (0, i, 0)),
            pl.BlockSpec((ROW_BLOCK, D_HID), lambda i: (i, 0)),
            pl.BlockSpec((1, 1, ROW_BLOCK), lambda i: (i, 0, 0)),
            pl.BlockSpec((1, D_HID), lambda i: (0, 0)),
            pl.BlockSpec((D_HID, D_OUT), lambda i: (0, 0)),
        ],
        out_specs=pl.BlockSpec((ROW_BLOCK, D_OUT), lambda i: (i, 0)),
        out_shape=jax.ShapeDtypeStruct((N_NODES, D_OUT), jnp.float32),
    )(acc1, xh1, dinv, b1, W2)


def _tc3_body(acc_ref, xh_ref, dinv_ref, b_ref, out_ref):
    out_ref[...] = ((acc_ref[0] + acc_ref[1] + xh_ref[...])
                    * _col(dinv_ref[...]) + b_ref[...])


def _tc3(acc2, xh2, dinv, b2):
    return pl.pallas_call(
        _tc3_body,
        grid=(N_ROW_BLOCKS,),
        in_specs=[
            pl.BlockSpec((NC, ROW_BLOCK, D_OUT), lambda i: (0, i, 0)),
            pl.BlockSpec((ROW_BLOCK, D_OUT), lambda i: (i, 0)),
            pl.BlockSpec((1, 1, ROW_BLOCK), lambda i: (i, 0, 0)),
            pl.BlockSpec((1, D_OUT), lambda i: (0, 0)),
        ],
        out_specs=pl.BlockSpec((ROW_BLOCK, D_OUT), lambda i: (i, 0)),
        out_shape=jax.ShapeDtypeStruct((N_NODES, D_OUT), jnp.float32),
    )(acc2, xh2, dinv, b2)


def kernel(x, edge_index, W1, b1, W2, b2):
    src2 = edge_index[0].astype(jnp.int32).reshape(NW, EPT)
    dst2 = edge_index[1].astype(jnp.int32).reshape(NW, EPT)

    cnt = _count_kernel(dst2)
    cnt0 = cnt[:N_NODES].reshape(N_ROW_BLOCKS, 1, ROW_BLOCK)
    cnt1 = cnt[N_PAD:N_PAD + N_NODES].reshape(N_ROW_BLOCKS, 1, ROW_BLOCK)
    xh1, dinv = _tc1(cnt0, cnt1, x, W1)
    acc1 = _scatter_hid(src2, dst2, xh1)
    xh2 = _tc2(acc1, xh1, dinv, b1.reshape(1, D_HID), W2)
    acc2 = _scatter_out(src2, dst2, xh2)
    return _tc3(acc2, xh2, dinv, b2.reshape(1, D_OUT))


# speedup vs baseline: 3.1323x; 1.0404x over previous
import functools

import jax
import jax.numpy as jnp
from jax import lax
from jax.experimental import pallas as pl
from jax.experimental.pallas import tpu as pltpu
from jax.experimental.pallas import tpu_sc as plsc

N_NODES = 10000
N_EDGES = 320000
D_IN = 128
D_HID = 64
D_OUT = 128

NC = 2
NS = 16
NW = NC * NS
EPT = N_EDGES // NW
N_PAD = 10240
ROWS_PER_TILE = N_PAD // NS
CHUNK_CNT = 400
CHUNK_HID = 400
CHUNK_OUT = 80
ROW_BLOCK = 1000
N_ROW_BLOCKS = N_NODES // ROW_BLOCK


def _mesh():
    return plsc.VectorSubcoreMesh(core_axis_name="c", subcore_axis_name="s",
                                  num_cores=NC, num_subcores=NS)


@functools.partial(
    pl.kernel,
    out_type=jax.ShapeDtypeStruct((NC * N_PAD,), jnp.float32),
    mesh=_mesh(),
    scratch_types=[
        pltpu.VMEM((EPT,), jnp.int32),
        pltpu.VMEM((CHUNK_CNT,), jnp.float32),
        pltpu.VMEM((ROWS_PER_TILE,), jnp.float32),
        pltpu.VMEM_SHARED((N_PAD,), jnp.float32),
    ],
    compiler_params=pltpu.CompilerParams(use_tc_tiling_on_sc=False),
)
def _count_kernel(edge_hbm, cnt_hbm, idx_v, ones_v, tmp_v, acc_s):
    c = lax.axis_index("c")
    s = lax.axis_index("s")
    wid = c * NS + s
    pltpu.sync_copy(edge_hbm.at[1, pl.ds(wid * EPT, EPT)], idx_v)

    def ones_body(i, carry):
        ones_v[pl.ds(i * 16, 16)] = jnp.ones((16,), jnp.float32)
        return carry

    lax.fori_loop(0, CHUNK_CNT // 16, ones_body, 0)

    def zero_body(i, carry):
        tmp_v[pl.ds(i * 16, 16)] = jnp.zeros((16,), jnp.float32)
        return carry

    lax.fori_loop(0, ROWS_PER_TILE // 16, zero_body, 0)
    tile_rows = pl.ds(s * ROWS_PER_TILE, ROWS_PER_TILE)
    pltpu.sync_copy(tmp_v, acc_s.at[tile_rows])
    plsc.subcore_barrier()

    def chunk_body(j, carry):
        pltpu.sync_copy(ones_v,
                        acc_s.at[idx_v.at[pl.ds(j * CHUNK_CNT, CHUNK_CNT)]],
                        add=True)
        return carry

    lax.fori_loop(0, EPT // CHUNK_CNT, chunk_body, 0)
    plsc.subcore_barrier()
    pltpu.sync_copy(acc_s.at[tile_rows], tmp_v)
    pltpu.sync_copy(tmp_v,
                    cnt_hbm.at[pl.ds(c * N_PAD + s * ROWS_PER_TILE,
                                     ROWS_PER_TILE)])


def _make_scatter(D, chunk):
    chunks = EPT // chunk
    assert EPT % chunk == 0 and chunks >= 3 and chunk % 8 == 0

    @functools.partial(
        pl.kernel,
        out_type=jax.ShapeDtypeStruct((NC, N_PAD, D), jnp.float32),
        mesh=_mesh(),
        scratch_types=[
            pltpu.VMEM((EPT,), jnp.int32),
            pltpu.VMEM((EPT,), jnp.int32),
            pltpu.VMEM((chunk, D), jnp.float32),
            pltpu.VMEM((chunk, D), jnp.float32),
            pltpu.VMEM_SHARED((N_PAD, D), jnp.float32),
            pltpu.SemaphoreType.DMA,
            pltpu.SemaphoreType.DMA,
        ],
        compiler_params=pltpu.CompilerParams(use_tc_tiling_on_sc=False),
    )
    def _scatter(edge_hbm, xh_hbm, out_hbm,
                 srcv, dstv, rows0, rows1, acc_s, sem0, sem1):
        c = lax.axis_index("c")
        s = lax.axis_index("s")
        wid = c * NS + s
        pltpu.sync_copy(edge_hbm.at[0, pl.ds(wid * EPT, EPT)], srcv)
        pltpu.sync_copy(edge_hbm.at[1, pl.ds(wid * EPT, EPT)], dstv)

        def sidx(ref, j):
            return ref.at[pl.ds(j * chunk, chunk)]

        def zero_body(r, carry):
            for i in range(D // 16):
                rows0[r, pl.ds(i * 16, 16)] = jnp.zeros((16,), jnp.float32)
            return carry

        lax.fori_loop(0, chunk, zero_body, 0)
        zchunks = ROWS_PER_TILE // chunk
        ztail = ROWS_PER_TILE % chunk

        def zslices():
            out = []
            for z in range(zchunks):
                out.append((pl.ds(s * ROWS_PER_TILE + z * chunk, chunk), None))
            if ztail:
                out.append((pl.ds(s * ROWS_PER_TILE + zchunks * chunk, ztail),
                            ztail))
            return out

        for sl, size in zslices():
            buf = rows0 if size is None else rows0.at[pl.ds(0, size)]
            pltpu.async_copy(buf, acc_s.at[sl], sem0)
        for sl, size in zslices():
            buf = rows0 if size is None else rows0.at[pl.ds(0, size)]
            pltpu.make_async_copy(buf, acc_s.at[sl], sem0).wait()
        plsc.subcore_barrier()

        pltpu.async_copy(xh_hbm.at[sidx(srcv, 0)], rows0, sem0)
        pltpu.async_copy(xh_hbm.at[sidx(srcv, 1)], rows1, sem1)

        def pair_body(p, carry):
            j0 = p * 2
            pltpu.make_async_copy(
                xh_hbm.at[sidx(srcv, j0)], rows0, sem0).wait()
            pltpu.sync_copy(rows0, acc_s.at[sidx(dstv, j0)], add=True)

            @pl.when(j0 + 2 < chunks)
            def _():
                pltpu.async_copy(xh_hbm.at[sidx(srcv, j0 + 2)], rows0, sem0)

            pltpu.make_async_copy(
                xh_hbm.at[sidx(srcv, j0 + 1)], rows1, sem1).wait()
            pltpu.sync_copy(rows1, acc_s.at[sidx(dstv, j0 + 1)], add=True)

            @pl.when(j0 + 3 < chunks)
            def _():
                pltpu.async_copy(xh_hbm.at[sidx(srcv, j0 + 3)], rows1, sem1)

            return carry

        lax.fori_loop(0, chunks // 2, pair_body, 0)
        if chunks % 2:
            jl = chunks - 1
            pltpu.make_async_copy(
                xh_hbm.at[sidx(srcv, jl)], rows0, sem0).wait()
            pltpu.sync_copy(rows0, acc_s.at[sidx(dstv, jl)], add=True)

        plsc.subcore_barrier()
        zs = zslices()
        for z, (sl, size) in enumerate(zs):
            buf = rows0 if z % 2 == 0 else rows1
            bufv = buf if size is None else buf.at[pl.ds(0, size)]
            pltpu.sync_copy(acc_s.at[sl], bufv)
            pltpu.sync_copy(bufv, out_hbm.at[c, sl])

    return _scatter


_scatter_hid = _make_scatter(D_HID, CHUNK_HID)
_scatter_out = _make_scatter(D_OUT, CHUNK_OUT)


def _col(v):
    return jnp.transpose(v.reshape(1, -1))


def _tc1_body(cnt0_ref, cnt1_ref, x_ref, w_ref, xh_ref, dinv_ref):
    cnt = cnt0_ref[...] + cnt1_ref[...]
    dinv = lax.rsqrt(cnt + 1.0)
    dinv_ref[...] = dinv
    h = jnp.dot(x_ref[...], w_ref[...], preferred_element_type=jnp.float32)
    xh_ref[...] = h * _col(dinv)


def _tc1(cnt0, cnt1, x, W1):
    return pl.pallas_call(
        _tc1_body,
        grid=(N_ROW_BLOCKS,),
        in_specs=[
            pl.BlockSpec((1, 1, ROW_BLOCK), lambda i: (i, 0, 0)),
            pl.BlockSpec((1, 1, ROW_BLOCK), lambda i: (i, 0, 0)),
            pl.BlockSpec((ROW_BLOCK, D_IN), lambda i: (i, 0)),
            pl.BlockSpec((D_IN, D_HID), lambda i: (0, 0)),
        ],
        out_specs=[
            pl.BlockSpec((ROW_BLOCK, D_HID), lambda i: (i, 0)),
            pl.BlockSpec((1, 1, ROW_BLOCK), lambda i: (i, 0, 0)),
        ],
        out_shape=[
            jax.ShapeDtypeStruct((N_NODES, D_HID), jnp.float32),
            jax.ShapeDtypeStruct((N_ROW_BLOCKS, 1, ROW_BLOCK), jnp.float32),
        ],
    )(cnt0, cnt1, x, W1)


def _tc2_body(acc_ref, xh_ref, dinv_ref, b_ref, w_ref, out_ref):
    dinv = _col(dinv_ref[...])
    z = (acc_ref[0] + acc_ref[1] + xh_ref[...]) * dinv + b_ref[...]
    a = jnp.maximum(z, 0.0)
    out_ref[...] = jnp.dot(
        a, w_ref[...], preferred_element_type=jnp.float32) * dinv


def _tc2(acc1, xh1, dinv, b1, W2):
    return pl.pallas_call(
        _tc2_body,
        grid=(N_ROW_BLOCKS,),
        in_specs=[
            pl.BlockSpec((NC, ROW_BLOCK, D_HID), lambda i: (0, i, 0)),
            pl.BlockSpec((ROW_BLOCK, D_HID), lambda i: (i, 0)),
            pl.BlockSpec((1, 1, ROW_BLOCK), lambda i: (i, 0, 0)),
            pl.BlockSpec((1, D_HID), lambda i: (0, 0)),
            pl.BlockSpec((D_HID, D_OUT), lambda i: (0, 0)),
        ],
        out_specs=pl.BlockSpec((ROW_BLOCK, D_OUT), lambda i: (i, 0)),
        out_shape=jax.ShapeDtypeStruct((N_NODES, D_OUT), jnp.float32),
    )(acc1, xh1, dinv, b1, W2)


def _tc3_body(acc_ref, xh_ref, dinv_ref, b_ref, out_ref):
    out_ref[...] = ((acc_ref[0] + acc_ref[1] + xh_ref[...])
                    * _col(dinv_ref[...]) + b_ref[...])


def _tc3(acc2, xh2, dinv, b2):
    return pl.pallas_call(
        _tc3_body,
        grid=(N_ROW_BLOCKS,),
        in_specs=[
            pl.BlockSpec((NC, ROW_BLOCK, D_OUT), lambda i: (0, i, 0)),
            pl.BlockSpec((ROW_BLOCK, D_OUT), lambda i: (i, 0)),
            pl.BlockSpec((1, 1, ROW_BLOCK), lambda i: (i, 0, 0)),
            pl.BlockSpec((1, D_OUT), lambda i: (0, 0)),
        ],
        out_specs=pl.BlockSpec((ROW_BLOCK, D_OUT), lambda i: (i, 0)),
        out_shape=jax.ShapeDtypeStruct((N_NODES, D_OUT), jnp.float32),
    )(acc2, xh2, dinv, b2)


def kernel(x, edge_index, W1, b1, W2, b2):
    edges = edge_index.astype(jnp.int32)

    cnt = _count_kernel(edges)
    cnt0 = cnt[:N_NODES].reshape(N_ROW_BLOCKS, 1, ROW_BLOCK)
    cnt1 = cnt[N_PAD:N_PAD + N_NODES].reshape(N_ROW_BLOCKS, 1, ROW_BLOCK)
    xh1, dinv = _tc1(cnt0, cnt1, x, W1)
    acc1 = _scatter_hid(edges, xh1)
    xh2 = _tc2(acc1, xh1, dinv, b1.reshape(1, D_HID), W2)
    acc2 = _scatter_out(edges, xh2)
    return _tc3(acc2, xh2, dinv, b2.reshape(1, D_OUT))
